# Initial kernel scaffold; baseline (speedup 1.0000x reference)
import jax
import jax.numpy as jnp
from jax.experimental import pallas as pl


def kernel(idx, table):
    raise NotImplementedError("write your pallas kernel here")


# speedup vs baseline: 1.6311x; 1.6311x over previous
import functools

import jax
import jax.numpy as jnp
from jax import lax
from jax.experimental import pallas as pl
from jax.experimental.pallas import tpu as pltpu
from jax.experimental.pallas import tpu_sc as plsc

VOCAB = 8192
D = 8192
B, T = 16, 512
N_IDX = B * T
NC, NS = 2, 16
NW = NC * NS
B_PER_W = N_IDX // NW
CH = 4
NCHUNK = B_PER_W // CH


def _gather_body(idx_hbm, table_hbm, out_hbm, idx_v, buf, gsem):
    wid = lax.axis_index("s") * NC + lax.axis_index("c")
    base = wid * B_PER_W
    pltpu.sync_copy(idx_hbm.at[wid], idx_v)

    @pl.loop(0, NCHUNK)
    def _chunk(g):
        pltpu.async_copy(table_hbm.at[idx_v.at[g]], buf, gsem).wait()
        pltpu.sync_copy(buf, out_hbm.at[pl.ds(base + g * CH, CH)])


@jax.jit
def _gather(idx_flat, table):
    mesh = plsc.VectorSubcoreMesh(
        core_axis_name="c", subcore_axis_name="s", num_cores=NC, num_subcores=NS
    )
    return pl.kernel(
        _gather_body,
        out_type=jax.ShapeDtypeStruct((N_IDX, D), jnp.float32),
        mesh=mesh,
        scratch_types=[
            pltpu.VMEM((NCHUNK, CH), jnp.int32),
            pltpu.VMEM((CH, D), jnp.float32),
            pltpu.SemaphoreType.DMA,
        ],
    )(idx_flat, table)


def kernel(idx, table):
    idx_w = idx.reshape(NW, NCHUNK, CH).astype(jnp.int32)
    out = _gather(idx_w, table)
    return out.reshape(B, T, D)


# speedup vs baseline: 1.9752x; 1.2109x over previous
import functools

import jax
import jax.numpy as jnp
from jax import lax
from jax.experimental import pallas as pl
from jax.experimental.pallas import tpu as pltpu
from jax.experimental.pallas import tpu_sc as plsc

VOCAB = 8192
D = 8192
B, T = 16, 512
N_IDX = B * T
NC, NS = 2, 16
NW = NC * NS
B_PER_W = N_IDX // NW
CH = 4
NCHUNK = B_PER_W // CH


def _gather_body(idx_hbm, table_hbm, out_hbm, idx_v, bufs, gsem, ssem):
    wid = lax.axis_index("s") * NC + lax.axis_index("c")
    base = wid * B_PER_W
    pltpu.sync_copy(idx_hbm.at[wid], idx_v)

    def gather_desc(g, b):
        return pltpu.make_async_copy(
            table_hbm.at[idx_v.at[g]], bufs.at[b], gsem.at[b]
        )

    def scatter_desc(g, b):
        return pltpu.make_async_copy(
            bufs.at[b], out_hbm.at[pl.ds(base + g * CH, CH)], ssem.at[b]
        )

    gather_desc(0, 0).start()

    @pl.loop(0, NCHUNK)
    def _chunk(g):
        b = lax.rem(g, 2)
        nb = lax.rem(g + 1, 2)

        @pl.when(g + 1 < NCHUNK)
        def _issue_next():
            @pl.when(g >= 1)
            def _drain_prev_scatter():
                scatter_desc(g - 1, nb).wait()

            gather_desc(g + 1, nb).start()

        gather_desc(g, b).wait()
        scatter_desc(g, b).start()

    scatter_desc(NCHUNK - 2, (NCHUNK - 2) % 2).wait()
    scatter_desc(NCHUNK - 1, (NCHUNK - 1) % 2).wait()


@jax.jit
def _gather(idx_flat, table):
    mesh = plsc.VectorSubcoreMesh(
        core_axis_name="c", subcore_axis_name="s", num_cores=NC, num_subcores=NS
    )
    return pl.kernel(
        _gather_body,
        out_type=jax.ShapeDtypeStruct((N_IDX, D), jnp.float32),
        mesh=mesh,
        scratch_types=[
            pltpu.VMEM((NCHUNK, CH), jnp.int32),
            pltpu.VMEM((2, CH, D), jnp.float32),
            pltpu.SemaphoreType.DMA((2,)),
            pltpu.SemaphoreType.DMA((2,)),
        ],
    )(idx_flat, table)


def kernel(idx, table):
    idx_w = idx.reshape(NW, NCHUNK, CH).astype(jnp.int32)
    out = _gather(idx_w, table)
    return out.reshape(B, T, D)
